# baseline (speedup 1.0000x reference)
import functools

import jax
import jax.numpy as jnp
from jax import lax
from jax.experimental import pallas as pl
from jax.experimental.pallas import tpu as pltpu
from jax.experimental.pallas import tpu_sc as plsc

E = 8
D = 2048
DFF = 4096
T = 256
NF = 8
BF = DFF // NF
NS = E * NF

_LANES = 16
_NTOK = T // _LANES


def _logits_kernel(xT_ref, wr_ref, br_ref, out_ref):
    out_ref[...] = jnp.dot(wr_ref[...], xT_ref[...],
                           preferred_element_type=jnp.float32) + br_ref[...]


def _router_logits(xT, Wr, br):
    return pl.pallas_call(
        _logits_kernel,
        in_specs=[
            pl.BlockSpec((D, T), lambda: (0, 0)),
            pl.BlockSpec((E, D), lambda: (0, 0)),
            pl.BlockSpec((E, 1), lambda: (0, 0)),
        ],
        out_specs=pl.BlockSpec((E, T), lambda: (0, 0)),
        out_shape=jax.ShapeDtypeStruct((E, T), jnp.float32),
    )(xT, Wr, br.reshape(E, 1))


def _sc_router(logits):
    mesh = plsc.VectorSubcoreMesh(core_axis_name="c", subcore_axis_name="s")
    info = plsc.get_sparse_core_info()
    nc = info.num_cores

    @functools.partial(
        pl.kernel, mesh=mesh,
        out_type=jax.ShapeDtypeStruct((E, T), jnp.float32),
        scratch_types=[
            pltpu.VMEM((E, _LANES), jnp.float32),
            pltpu.VMEM((E, _LANES), jnp.float32),
        ],
    )
    def k(logits_hbm, out_hbm, lg_v, w_v):
        wid = lax.axis_index("s") * nc + lax.axis_index("c")

        @pl.when(wid < _NTOK)
        def _():
            base = wid * _LANES
            for e in range(E):
                pltpu.sync_copy(logits_hbm.at[e, pl.ds(base, _LANES)],
                                lg_v.at[e])
            m1 = lg_v[0]
            i1 = jnp.zeros((_LANES,), jnp.float32)
            for e in range(1, E):
                v = lg_v[e]
                gt = v > m1
                m1 = jnp.where(gt, v, m1)
                i1 = jnp.where(gt, float(e), i1)
            m2 = jnp.where(i1 == 0.0, -jnp.inf, lg_v[0])
            i2 = jnp.zeros((_LANES,), jnp.float32)
            for e in range(1, E):
                v = jnp.where(i1 == float(e), -jnp.inf, lg_v[e])
                gt = v > m2
                m2 = jnp.where(gt, v, m2)
                i2 = jnp.where(gt, float(e), i2)
            p1 = 1.0 / (1.0 + jnp.exp(m2 - m1))
            p2 = 1.0 - p1
            for e in range(E):
                fe = float(e)
                w_v[e] = (jnp.where(i1 == fe, p1, 0.0)
                          + jnp.where(i2 == fe, p2, 0.0))
            for e in range(E):
                pltpu.sync_copy(w_v.at[e], out_hbm.at[e, pl.ds(base, _LANES)])

    return k(logits)


def _moe_kernel(xT_ref, w_ref, w1_ref, w2_ref, w3_ref,
                b1_ref, b2_ref, b3_ref, out_ref, xb_ref, h_ref):
    s = pl.program_id(0)

    @pl.when(s == 0)
    def _init():
        xb_ref[...] = xT_ref[...].astype(jnp.bfloat16)

    @pl.when(s < NS)
    def _h_phase():
        f = s % NF
        xb = xb_ref[...]
        b1f = b1_ref[0, pl.ds(f * BF, BF), :]
        b2f = b2_ref[0, pl.ds(f * BF, BF), :]
        h1 = jnp.dot(w1_ref[0], xb, preferred_element_type=jnp.float32) + b1f
        h2 = jnp.dot(w2_ref[0], xb, preferred_element_type=jnp.float32) + b2f
        h = h2 * (h1 * jax.nn.sigmoid(h1))
        h_ref[s % 2] = h.astype(jnp.bfloat16)

    @pl.when(s > 0)
    def _y_phase():
        sp = s - 1
        ep = sp // NF
        fp = sp % NF
        hp = h_ref[(s - 1) % 2]
        yblk = jnp.dot(w3_ref[0], hp, preferred_element_type=jnp.float32)
        wrow = w_ref[pl.ds(ep, 1), :]
        contrib = yblk * wrow

        @pl.when(fp == 0)
        def _b3():
            contrib2 = contrib + b3_ref[0] * wrow

            @pl.when(ep == 0)
            def _set():
                out_ref[...] = contrib2

            @pl.when(ep > 0)
            def _acc():
                out_ref[...] += contrib2

        @pl.when(fp > 0)
        def _nob3():
            out_ref[...] += contrib


def kernel(x, Wr, br, W1, b1, W2, b2, W3, b3):
    b, s_, d = x.shape
    xT = x.reshape(b * s_, d).T
    logits = _router_logits(xT, Wr, br)
    w = _sc_router(logits)
    last = NS - 1

    def w12_map(s):
        sc = jnp.minimum(s, last)
        return (sc // NF, sc % NF, 0)

    def w3_map(s):
        sp = jnp.maximum(s - 1, 0)
        return (sp // NF, 0, sp % NF)

    outT = pl.pallas_call(
        _moe_kernel,
        grid=(NS + 1,),
        in_specs=[
            pl.BlockSpec((D, T), lambda s: (0, 0)),
            pl.BlockSpec((E, T), lambda s: (0, 0)),
            pl.BlockSpec((1, BF, D), w12_map),
            pl.BlockSpec((1, BF, D), w12_map),
            pl.BlockSpec((1, D, BF), w3_map),
            pl.BlockSpec((1, DFF, 1),
                         lambda s: (jnp.minimum(s, last) // NF, 0, 0)),
            pl.BlockSpec((1, DFF, 1),
                         lambda s: (jnp.minimum(s, last) // NF, 0, 0)),
            pl.BlockSpec((1, D, 1),
                         lambda s: (jnp.maximum(s - 1, 0) // NF, 0, 0)),
        ],
        out_specs=pl.BlockSpec((D, T), lambda s: (0, 0)),
        out_shape=jax.ShapeDtypeStruct((D, T), jnp.float32),
        scratch_shapes=[
            pltpu.VMEM((D, T), jnp.bfloat16),
            pltpu.VMEM((2, BF, T), jnp.bfloat16),
        ],
    )(xT, w, W1, W2, W3,
      b1.reshape(E, DFF, 1), b2.reshape(E, DFF, 1), b3.reshape(E, D, 1))
    return outT.T.reshape(b, s_, d)


# speedup vs baseline: 1.0032x; 1.0032x over previous
import functools

import jax
import jax.numpy as jnp
from jax import lax
from jax.experimental import pallas as pl
from jax.experimental.pallas import tpu as pltpu
from jax.experimental.pallas import tpu_sc as plsc

E = 8
D = 2048
DFF = 4096
T = 256
NF = 8
BF = DFF // NF
NS = E * NF

_LANES = 16
_NTOK = T // _LANES


def _logits_kernel(xT_ref, wr_ref, br_ref, out_ref):
    out_ref[...] = jnp.dot(wr_ref[...], xT_ref[...],
                           preferred_element_type=jnp.float32) + br_ref[...]


def _router_logits(xT, Wr, br):
    return pl.pallas_call(
        _logits_kernel,
        in_specs=[
            pl.BlockSpec((D, T), lambda: (0, 0)),
            pl.BlockSpec((E, D), lambda: (0, 0)),
            pl.BlockSpec((E, 1), lambda: (0, 0)),
        ],
        out_specs=pl.BlockSpec((E, T), lambda: (0, 0)),
        out_shape=jax.ShapeDtypeStruct((E, T), jnp.float32),
    )(xT, Wr, br.reshape(E, 1))


def _sc_router(logits):
    mesh = plsc.VectorSubcoreMesh(core_axis_name="c", subcore_axis_name="s")
    info = plsc.get_sparse_core_info()
    nc = info.num_cores

    @functools.partial(
        pl.kernel, mesh=mesh,
        out_type=jax.ShapeDtypeStruct((E, T), jnp.float32),
        scratch_types=[
            pltpu.VMEM((E, _LANES), jnp.float32),
            pltpu.VMEM((E, _LANES), jnp.float32),
            pltpu.SemaphoreType.DMA,
        ],
    )
    def k(logits_hbm, out_hbm, lg_v, w_v, sem):
        wid = lax.axis_index("s") * nc + lax.axis_index("c")

        @pl.when(wid < _NTOK)
        def _():
            base = wid * _LANES
            hs = [pltpu.async_copy(logits_hbm.at[e, pl.ds(base, _LANES)],
                                   lg_v.at[e], sem) for e in range(E)]
            for hcp in hs:
                hcp.wait()
            m1 = lg_v[0]
            i1 = jnp.zeros((_LANES,), jnp.float32)
            for e in range(1, E):
                v = lg_v[e]
                gt = v > m1
                m1 = jnp.where(gt, v, m1)
                i1 = jnp.where(gt, float(e), i1)
            m2 = jnp.where(i1 == 0.0, -jnp.inf, lg_v[0])
            i2 = jnp.zeros((_LANES,), jnp.float32)
            for e in range(1, E):
                v = jnp.where(i1 == float(e), -jnp.inf, lg_v[e])
                gt = v > m2
                m2 = jnp.where(gt, v, m2)
                i2 = jnp.where(gt, float(e), i2)
            p1 = 1.0 / (1.0 + jnp.exp(m2 - m1))
            p2 = 1.0 - p1
            for e in range(E):
                fe = float(e)
                w_v[e] = (jnp.where(i1 == fe, p1, 0.0)
                          + jnp.where(i2 == fe, p2, 0.0))
            ho = [pltpu.async_copy(w_v.at[e],
                                   out_hbm.at[e, pl.ds(base, _LANES)], sem)
                  for e in range(E)]
            for hcp in ho:
                hcp.wait()

    return k(logits)


def _moe_kernel(xT_ref, w_ref, w1_ref, w2_ref, w3_ref,
                b1_ref, b2_ref, b3_ref, out_ref, xb_ref, h_ref):
    s = pl.program_id(0)

    @pl.when(s == 0)
    def _init():
        xb_ref[...] = xT_ref[...].astype(jnp.bfloat16)

    @pl.when(s < NS)
    def _h_phase():
        f = s % NF
        xb = xb_ref[...]
        b1f = b1_ref[0, pl.ds(f * BF, BF), :]
        b2f = b2_ref[0, pl.ds(f * BF, BF), :]
        h1 = jnp.dot(w1_ref[0], xb, preferred_element_type=jnp.float32) + b1f
        h2 = jnp.dot(w2_ref[0], xb, preferred_element_type=jnp.float32) + b2f
        h = h2 * (h1 * jax.nn.sigmoid(h1))
        h_ref[s % 2] = h.astype(jnp.bfloat16)

    @pl.when(s > 0)
    def _y_phase():
        sp = s - 1
        ep = sp // NF
        fp = sp % NF
        hp = h_ref[(s - 1) % 2]
        yblk = jnp.dot(w3_ref[0], hp, preferred_element_type=jnp.float32)
        wrow = w_ref[pl.ds(ep, 1), :]
        contrib = yblk * wrow

        @pl.when(fp == 0)
        def _b3():
            contrib2 = contrib + b3_ref[0] * wrow

            @pl.when(ep == 0)
            def _set():
                out_ref[...] = contrib2

            @pl.when(ep > 0)
            def _acc():
                out_ref[...] += contrib2

        @pl.when(fp > 0)
        def _nob3():
            out_ref[...] += contrib


def kernel(x, Wr, br, W1, b1, W2, b2, W3, b3):
    b, s_, d = x.shape
    xT = x.reshape(b * s_, d).T
    logits = _router_logits(xT, Wr, br)
    w = _sc_router(logits)
    last = NS - 1

    def w12_map(s):
        sc = jnp.minimum(s, last)
        return (sc // NF, sc % NF, 0)

    def w3_map(s):
        sp = jnp.maximum(s - 1, 0)
        return (sp // NF, 0, sp % NF)

    outT = pl.pallas_call(
        _moe_kernel,
        grid=(NS + 1,),
        in_specs=[
            pl.BlockSpec((D, T), lambda s: (0, 0)),
            pl.BlockSpec((E, T), lambda s: (0, 0)),
            pl.BlockSpec((1, BF, D), w12_map),
            pl.BlockSpec((1, BF, D), w12_map),
            pl.BlockSpec((1, D, BF), w3_map),
            pl.BlockSpec((1, DFF, 1),
                         lambda s: (jnp.minimum(s, last) // NF, 0, 0)),
            pl.BlockSpec((1, DFF, 1),
                         lambda s: (jnp.minimum(s, last) // NF, 0, 0)),
            pl.BlockSpec((1, D, 1),
                         lambda s: (jnp.maximum(s - 1, 0) // NF, 0, 0)),
        ],
        out_specs=pl.BlockSpec((D, T), lambda s: (0, 0)),
        out_shape=jax.ShapeDtypeStruct((D, T), jnp.float32),
        scratch_shapes=[
            pltpu.VMEM((D, T), jnp.bfloat16),
            pltpu.VMEM((2, BF, T), jnp.bfloat16),
        ],
    )(xT, w, W1, W2, W3,
      b1.reshape(E, DFF, 1), b2.reshape(E, DFF, 1), b3.reshape(E, D, 1))
    return outT.T.reshape(b, s_, d)


# speedup vs baseline: 1.0523x; 1.0490x over previous
import jax
import jax.numpy as jnp
from jax.experimental import pallas as pl
from jax.experimental.pallas import tpu as pltpu

E = 8
D = 2048
DFF = 4096
T = 256
NF = 8
BF = DFF // NF
BD = D // NF


def _moe_kernel(xT_ref, wr_ref, br_ref, w1_ref, w2_ref, w3_ref,
                b1_ref, b2_ref, b3_ref, out_ref,
                wrow_ref, h_ref, yacc_ref):
    e = pl.program_id(0)
    f = pl.program_id(1)

    @pl.when((e == 0) & (f == 0))
    def _init():
        logits = jnp.dot(wr_ref[...], xT_ref[...],
                         preferred_element_type=jnp.float32) + br_ref[...]
        idx = jax.lax.broadcasted_iota(jnp.int32, (E, T), 0)
        m1 = jnp.max(logits, axis=0, keepdims=True)
        i1 = jnp.min(jnp.where(logits == m1, idx, E), axis=0, keepdims=True)
        sel1 = idx == i1
        masked = jnp.where(sel1, -jnp.inf, logits)
        m2 = jnp.max(masked, axis=0, keepdims=True)
        i2 = jnp.min(jnp.where(masked == m2, idx, E), axis=0, keepdims=True)
        sel2 = idx == i2
        p1 = 1.0 / (1.0 + jnp.exp(m2 - m1))
        wrow_ref[...] = jnp.where(sel1, p1, 0.0) + jnp.where(sel2, 1.0 - p1, 0.0)

    @pl.when(e < E)
    def _h_phase():
        xv = xT_ref[...]
        b1f = b1_ref[0, pl.ds(f * BF, BF), :]
        b2f = b2_ref[0, pl.ds(f * BF, BF), :]
        h1 = jnp.dot(w1_ref[0], xv, preferred_element_type=jnp.float32) + b1f
        h2 = jnp.dot(w2_ref[0], xv, preferred_element_type=jnp.float32) + b2f
        h = h2 * (h1 * jax.nn.sigmoid(h1))
        h_ref[e % 2, pl.ds(f * BF, BF), :] = h.astype(jnp.bfloat16)

    @pl.when(e > 0)
    def _y_phase():
        ep = e - 1
        hprev = h_ref[(e - 1) % 2]
        yblk = jnp.dot(w3_ref[0], hprev, preferred_element_type=jnp.float32)
        wrow = wrow_ref[pl.ds(ep, 1), :]
        b3f = b3_ref[0, pl.ds(f * BD, BD), :]
        contrib = (yblk + b3f) * wrow

        @pl.when(ep == 0)
        def _first():
            yacc_ref[pl.ds(f * BD, BD), :] = contrib

        @pl.when(ep > 0)
        def _rest():
            yacc_ref[pl.ds(f * BD, BD), :] += contrib

        @pl.when((e == E) & (f == NF - 1))
        def _emit():
            out_ref[...] = yacc_ref[...]


def kernel(x, Wr, br, W1, b1, W2, b2, W3, b3):
    b, s, d = x.shape
    xT = x.reshape(b * s, d).T
    last = E - 1

    def w12_map(e, f):
        ec = jnp.minimum(e, last)
        fc = jnp.where(e == E, NF - 1, f)
        return (ec, fc, 0)

    def w3_map(e, f):
        ep = jnp.maximum(e - 1, 0)
        fc = jnp.where(e == 0, 0, f)
        return (ep, fc, 0)

    outT = pl.pallas_call(
        _moe_kernel,
        grid=(E + 1, NF),
        in_specs=[
            pl.BlockSpec((D, T), lambda e, f: (0, 0)),
            pl.BlockSpec((E, D), lambda e, f: (0, 0)),
            pl.BlockSpec((E, 1), lambda e, f: (0, 0)),
            pl.BlockSpec((1, BF, D), w12_map),
            pl.BlockSpec((1, BF, D), w12_map),
            pl.BlockSpec((1, BD, DFF), w3_map),
            pl.BlockSpec((1, DFF, 1), lambda e, f: (jnp.minimum(e, last), 0, 0)),
            pl.BlockSpec((1, DFF, 1), lambda e, f: (jnp.minimum(e, last), 0, 0)),
            pl.BlockSpec((1, D, 1), lambda e, f: (jnp.maximum(e - 1, 0), 0, 0)),
        ],
        out_specs=pl.BlockSpec((D, T), lambda e, f: (0, 0)),
        out_shape=jax.ShapeDtypeStruct((D, T), jnp.float32),
        scratch_shapes=[
            pltpu.VMEM((E, T), jnp.float32),
            pltpu.VMEM((2, DFF, T), jnp.bfloat16),
            pltpu.VMEM((D, T), jnp.float32),
        ],
    )(xT, Wr, br.reshape(E, 1), W1, W2, W3,
      b1.reshape(E, DFF, 1), b2.reshape(E, DFF, 1), b3.reshape(E, D, 1))
    return outT.T.reshape(b, s, d)
